# baseline (speedup 1.0000x reference)
import functools

import jax
import jax.numpy as jnp
from jax import lax
from jax.experimental import pallas as pl
from jax.experimental.pallas import tpu as pltpu
from jax.experimental.pallas import tpu_sc as plsc

_NUM_CORES = 2
_NUM_SUBCORES = 16
_NUM_WORKERS = _NUM_CORES * _NUM_SUBCORES

_TBLK = 8
_NBUF = 2


def _sc_gather(table, idx_pad, rows_per_worker):
    n_pad = idx_pad.shape[0]
    d = table.shape[1]
    mesh = plsc.VectorSubcoreMesh(core_axis_name="c", subcore_axis_name="s")

    @functools.partial(
        pl.kernel,
        mesh=mesh,
        out_type=jax.ShapeDtypeStruct((n_pad, d), jnp.float32),
        scratch_types=[
            pltpu.VMEM((rows_per_worker,), jnp.int32),
            pltpu.VMEM((rows_per_worker, d), jnp.float32),
            pltpu.SemaphoreType.DMA,
        ],
    )
    def gather_kernel(table_hbm, idx_hbm, out_hbm, idx_v, rows_v, sem):
        wid = lax.axis_index("s") * _NUM_CORES + lax.axis_index("c")
        base = wid * rows_per_worker
        pltpu.sync_copy(idx_hbm.at[pl.ds(base, rows_per_worker)], idx_v)
        pltpu.async_copy(table_hbm.at[idx_v], rows_v, sem).wait()
        pltpu.sync_copy(rows_v, out_hbm.at[pl.ds(base, rows_per_worker)])

    return gather_kernel(table, idx_pad)


def _make_add_body(b, t, p, d):
    t_chunks = t // _TBLK
    nsteps = t_chunks

    def add_body(pe_ref, x_hbm, o_hbm, ibuf, obuf, isem, osem):
        s = pl.program_id(0)

        def in_copy(step, slot):
            t0 = (step % t_chunks) * _TBLK
            return pltpu.make_async_copy(
                x_hbm.at[:, pl.ds(t0, _TBLK)], ibuf.at[slot], isem.at[slot])

        def out_copy(step, slot):
            t0 = (step % t_chunks) * _TBLK
            return pltpu.make_async_copy(
                obuf.at[slot], o_hbm.at[:, pl.ds(t0, _TBLK)], osem.at[slot])

        @pl.when(s == 0)
        def _prologue():
            for k in range(_NBUF):
                in_copy(k, k).start()

        slot = jax.lax.rem(s, _NBUF)
        in_copy(s, slot).wait()

        @pl.when(s >= _NBUF)
        def _():
            out_copy(s - _NBUF, slot).wait()

        pe_blk = pe_ref[:, pl.ds(s * _TBLK, _TBLK), :]
        obuf[slot] = ibuf[slot] + pe_blk[:, :, None, :]
        out_copy(s, slot).start()

        @pl.when(s + _NBUF < nsteps)
        def _():
            in_copy(s + _NBUF, slot).start()

        @pl.when(s == nsteps - 1)
        def _epilogue():
            for k in range(_NBUF):
                step = nsteps - _NBUF + k
                out_copy(step, step % _NBUF).wait()

    return add_body, nsteps


def kernel(x, frame_indices, frame_embed):
    b, t, p, d = x.shape
    bt = b * t

    align = 8 * _NUM_WORKERS
    bt_pad = ((bt + align - 1) // align) * align
    idx = frame_indices.reshape(bt).astype(jnp.int32)
    idx_pad = jnp.pad(idx, (0, bt_pad - bt))

    pe = _sc_gather(frame_embed, idx_pad, bt_pad // _NUM_WORKERS)
    pe3 = pe[:bt].reshape(b, t, d)

    add_body, nsteps = _make_add_body(b, t, p, d)
    out = pl.pallas_call(
        add_body,
        grid=(nsteps,),
        in_specs=[
            pl.BlockSpec((b, t, d), lambda s: (0, 0, 0)),
            pl.BlockSpec(memory_space=pl.ANY),
        ],
        out_specs=pl.BlockSpec(memory_space=pl.ANY),
        out_shape=jax.ShapeDtypeStruct((b, t, p, d), jnp.float32),
        scratch_shapes=[
            pltpu.VMEM((_NBUF, b, _TBLK, p, d), jnp.float32),
            pltpu.VMEM((_NBUF, b, _TBLK, p, d), jnp.float32),
            pltpu.SemaphoreType.DMA((_NBUF,)),
            pltpu.SemaphoreType.DMA((_NBUF,)),
        ],
        compiler_params=pltpu.CompilerParams(
            dimension_semantics=("arbitrary",),
            vmem_limit_bytes=100 * 1024 * 1024,
        ),
    )(pe3, x)
    return out


# speedup vs baseline: 2.7687x; 2.7687x over previous
import functools

import jax
import jax.numpy as jnp
from jax import lax
from jax.experimental import pallas as pl
from jax.experimental.pallas import tpu as pltpu
from jax.experimental.pallas import tpu_sc as plsc

_NUM_CORES = 2
_NUM_SUBCORES = 16
_NUM_WORKERS = _NUM_CORES * _NUM_SUBCORES

_PCH = 7
_NBUF = 2


def _sc_gather(table, idx_pad, rows_per_worker):
    n_pad = idx_pad.shape[0]
    d = table.shape[1]
    mesh = plsc.VectorSubcoreMesh(core_axis_name="c", subcore_axis_name="s")

    @functools.partial(
        pl.kernel,
        mesh=mesh,
        out_type=jax.ShapeDtypeStruct((n_pad, d), jnp.float32),
        scratch_types=[
            pltpu.VMEM((rows_per_worker,), jnp.int32),
            pltpu.VMEM((rows_per_worker, d), jnp.float32),
            pltpu.SemaphoreType.DMA,
        ],
    )
    def gather_kernel(table_hbm, idx_hbm, out_hbm, idx_v, rows_v, sem):
        wid = lax.axis_index("s") * _NUM_CORES + lax.axis_index("c")
        base = wid * rows_per_worker
        pltpu.sync_copy(idx_hbm.at[pl.ds(base, rows_per_worker)], idx_v)
        pltpu.async_copy(table_hbm.at[idx_v], rows_v, sem).wait()
        pltpu.sync_copy(rows_v, out_hbm.at[pl.ds(base, rows_per_worker)])

    return gather_kernel(table, idx_pad)


def _make_add_body(b, t, p, d):
    nsteps = p // _PCH

    def add_body(pe_ref, x_hbm, o_hbm, ibuf, obuf, isem, osem):
        s = pl.program_id(0)

        def in_copy(step, slot):
            return pltpu.make_async_copy(
                x_hbm.at[:, pl.ds(step * _PCH, _PCH)], ibuf.at[slot],
                isem.at[slot])

        def out_copy(step, slot):
            return pltpu.make_async_copy(
                obuf.at[slot], o_hbm.at[:, pl.ds(step * _PCH, _PCH)],
                osem.at[slot])

        @pl.when(s == 0)
        def _prologue():
            for k in range(_NBUF):
                in_copy(k, k).start()

        slot = jax.lax.rem(s, _NBUF)
        in_copy(s, slot).wait()

        @pl.when(s >= _NBUF)
        def _():
            out_copy(s - _NBUF, slot).wait()

        obuf[slot] = ibuf[slot] + pe_ref[...][:, None, :, :]
        out_copy(s, slot).start()

        @pl.when(s + _NBUF < nsteps)
        def _():
            in_copy(s + _NBUF, slot).start()

        @pl.when(s == nsteps - 1)
        def _epilogue():
            for k in range(_NBUF):
                step = nsteps - _NBUF + k
                out_copy(step, step % _NBUF).wait()

    return add_body, nsteps


def kernel(x, frame_indices, frame_embed):
    b, t, p, d = x.shape
    bt = b * t

    align = 8 * _NUM_WORKERS
    bt_pad = ((bt + align - 1) // align) * align
    idx = frame_indices.reshape(bt).astype(jnp.int32)
    idx_pad = jnp.pad(idx, (0, bt_pad - bt))

    pe = _sc_gather(frame_embed, idx_pad, bt_pad // _NUM_WORKERS)
    pe3 = pe[:bt].reshape(b, t, d)

    xt = jnp.swapaxes(x, 1, 2)
    add_body, nsteps = _make_add_body(b, t, p, d)
    out_t = pl.pallas_call(
        add_body,
        grid=(nsteps,),
        in_specs=[
            pl.BlockSpec((b, t, d), lambda s: (0, 0, 0)),
            pl.BlockSpec(memory_space=pl.ANY),
        ],
        out_specs=pl.BlockSpec(memory_space=pl.ANY),
        out_shape=jax.ShapeDtypeStruct((b, p, t, d), jnp.float32),
        scratch_shapes=[
            pltpu.VMEM((_NBUF, b, _PCH, t, d), jnp.float32),
            pltpu.VMEM((_NBUF, b, _PCH, t, d), jnp.float32),
            pltpu.SemaphoreType.DMA((_NBUF,)),
            pltpu.SemaphoreType.DMA((_NBUF,)),
        ],
        compiler_params=pltpu.CompilerParams(
            dimension_semantics=("arbitrary",),
            vmem_limit_bytes=63 * 1024 * 1024,
        ),
    )(pe3, xt)
    return jnp.swapaxes(out_t, 1, 2)


# speedup vs baseline: 2.7957x; 1.0098x over previous
import functools

import jax
import jax.numpy as jnp
from jax import lax
from jax.experimental import pallas as pl
from jax.experimental.pallas import tpu as pltpu
from jax.experimental.pallas import tpu_sc as plsc

_NUM_CORES = 2
_NUM_SUBCORES = 16
_NUM_WORKERS = _NUM_CORES * _NUM_SUBCORES

_PCH = 7
_NBUF = 2


def _sc_gather(table, idx_pad, rows_per_worker):
    n_pad = idx_pad.shape[0]
    d = table.shape[1]
    mesh = plsc.VectorSubcoreMesh(core_axis_name="c", subcore_axis_name="s")

    h0 = (rows_per_worker // 2 + 7) // 8 * 8
    h1 = rows_per_worker - h0

    @functools.partial(
        pl.kernel,
        mesh=mesh,
        out_type=jax.ShapeDtypeStruct((n_pad, d), jnp.float32),
        scratch_types=[
            pltpu.VMEM((rows_per_worker,), jnp.int32),
            pltpu.VMEM((rows_per_worker, d), jnp.float32),
            pltpu.SemaphoreType.DMA,
            pltpu.SemaphoreType.DMA,
            pltpu.SemaphoreType.DMA,
        ],
    )
    def gather_kernel(table_hbm, idx_hbm, out_hbm, idx_v, rows_v, g0, g1, w0):
        wid = lax.axis_index("s") * _NUM_CORES + lax.axis_index("c")
        base = wid * rows_per_worker
        pltpu.sync_copy(idx_hbm.at[pl.ds(base, rows_per_worker)], idx_v)
        c0 = pltpu.make_async_copy(
            table_hbm.at[idx_v.at[pl.ds(0, h0)]], rows_v.at[pl.ds(0, h0)], g0)
        c1 = pltpu.make_async_copy(
            table_hbm.at[idx_v.at[pl.ds(h0, h1)]], rows_v.at[pl.ds(h0, h1)], g1)
        c0.start()
        c1.start()
        c0.wait()
        wb0 = pltpu.make_async_copy(
            rows_v.at[pl.ds(0, h0)], out_hbm.at[pl.ds(base, h0)], w0)
        wb0.start()
        c1.wait()
        pltpu.sync_copy(
            rows_v.at[pl.ds(h0, h1)], out_hbm.at[pl.ds(base + h0, h1)])
        wb0.wait()

    return gather_kernel(table, idx_pad)


def _make_add_body(b, t, p, d):
    nsteps = p // _PCH

    def add_body(pe_ref, x_hbm, o_hbm, ibuf, obuf, isem, osem):
        s = pl.program_id(0)

        def in_copy(step, slot):
            return pltpu.make_async_copy(
                x_hbm.at[:, pl.ds(step * _PCH, _PCH)], ibuf.at[slot],
                isem.at[slot])

        def out_copy(step, slot):
            return pltpu.make_async_copy(
                obuf.at[slot], o_hbm.at[:, pl.ds(step * _PCH, _PCH)],
                osem.at[slot])

        @pl.when(s == 0)
        def _prologue():
            for k in range(_NBUF):
                in_copy(k, k).start()

        slot = jax.lax.rem(s, _NBUF)
        in_copy(s, slot).wait()

        @pl.when(s >= _NBUF)
        def _():
            out_copy(s - _NBUF, slot).wait()

        pe3 = pe_ref[pl.ds(0, b * t), :].reshape(b, t, d)
        obuf[slot] = ibuf[slot] + pe3[:, None, :, :]
        out_copy(s, slot).start()

        @pl.when(s + _NBUF < nsteps)
        def _():
            in_copy(s + _NBUF, slot).start()

        @pl.when(s == nsteps - 1)
        def _epilogue():
            for k in range(_NBUF):
                step = nsteps - _NBUF + k
                out_copy(step, step % _NBUF).wait()

    return add_body, nsteps


def kernel(x, frame_indices, frame_embed):
    b, t, p, d = x.shape
    bt = b * t

    align = 8 * _NUM_WORKERS
    bt_pad = ((bt + align - 1) // align) * align
    idx = frame_indices.reshape(bt).astype(jnp.int32)
    idx_pad = jnp.pad(idx, (0, bt_pad - bt))

    pe = _sc_gather(frame_embed, idx_pad, bt_pad // _NUM_WORKERS)

    xt = jnp.swapaxes(x, 1, 2)
    add_body, nsteps = _make_add_body(b, t, p, d)
    out_t = pl.pallas_call(
        add_body,
        grid=(nsteps,),
        in_specs=[
            pl.BlockSpec((bt_pad, d), lambda s: (0, 0)),
            pl.BlockSpec(memory_space=pl.ANY),
        ],
        out_specs=pl.BlockSpec(memory_space=pl.ANY),
        out_shape=jax.ShapeDtypeStruct((b, p, t, d), jnp.float32),
        scratch_shapes=[
            pltpu.VMEM((_NBUF, b, _PCH, t, d), jnp.float32),
            pltpu.VMEM((_NBUF, b, _PCH, t, d), jnp.float32),
            pltpu.SemaphoreType.DMA((_NBUF,)),
            pltpu.SemaphoreType.DMA((_NBUF,)),
        ],
        compiler_params=pltpu.CompilerParams(
            dimension_semantics=("arbitrary",),
            vmem_limit_bytes=63 * 1024 * 1024,
        ),
    )(pe, xt)
    return jnp.swapaxes(out_t, 1, 2)


# speedup vs baseline: 2.7970x; 1.0005x over previous
import functools

import jax
import jax.numpy as jnp
from jax import lax
from jax.experimental import pallas as pl
from jax.experimental.pallas import tpu as pltpu
from jax.experimental.pallas import tpu_sc as plsc

_NUM_CORES = 2
_NUM_SUBCORES = 16
_NUM_WORKERS = _NUM_CORES * _NUM_SUBCORES

_PCH = 4
_NBUF = 3


def _sc_gather(table, idx_pad, rows_per_worker):
    n_pad = idx_pad.shape[0]
    d = table.shape[1]
    mesh = plsc.VectorSubcoreMesh(core_axis_name="c", subcore_axis_name="s")

    h0 = (rows_per_worker // 2 + 7) // 8 * 8
    h1 = rows_per_worker - h0

    @functools.partial(
        pl.kernel,
        mesh=mesh,
        out_type=jax.ShapeDtypeStruct((n_pad, d), jnp.float32),
        scratch_types=[
            pltpu.VMEM((rows_per_worker,), jnp.int32),
            pltpu.VMEM((rows_per_worker, d), jnp.float32),
            pltpu.SemaphoreType.DMA,
            pltpu.SemaphoreType.DMA,
            pltpu.SemaphoreType.DMA,
        ],
    )
    def gather_kernel(table_hbm, idx_hbm, out_hbm, idx_v, rows_v, g0, g1, w0):
        wid = lax.axis_index("s") * _NUM_CORES + lax.axis_index("c")
        base = wid * rows_per_worker
        pltpu.sync_copy(idx_hbm.at[pl.ds(base, rows_per_worker)], idx_v)
        c0 = pltpu.make_async_copy(
            table_hbm.at[idx_v.at[pl.ds(0, h0)]], rows_v.at[pl.ds(0, h0)], g0)
        c1 = pltpu.make_async_copy(
            table_hbm.at[idx_v.at[pl.ds(h0, h1)]], rows_v.at[pl.ds(h0, h1)], g1)
        c0.start()
        c1.start()
        c0.wait()
        wb0 = pltpu.make_async_copy(
            rows_v.at[pl.ds(0, h0)], out_hbm.at[pl.ds(base, h0)], w0)
        wb0.start()
        c1.wait()
        pltpu.sync_copy(
            rows_v.at[pl.ds(h0, h1)], out_hbm.at[pl.ds(base + h0, h1)])
        wb0.wait()

    return gather_kernel(table, idx_pad)


def _make_add_body(b, t, p, d):
    nsteps = p // _PCH

    def add_body(pe_ref, x_hbm, o_hbm, ibuf, obuf, isem, osem):
        s = pl.program_id(0)

        def in_copy(step, slot):
            return pltpu.make_async_copy(
                x_hbm.at[:, pl.ds(step * _PCH, _PCH)], ibuf.at[slot],
                isem.at[slot])

        def out_copy(step, slot):
            return pltpu.make_async_copy(
                obuf.at[slot], o_hbm.at[:, pl.ds(step * _PCH, _PCH)],
                osem.at[slot])

        @pl.when(s == 0)
        def _prologue():
            for k in range(_NBUF):
                in_copy(k, k).start()

        slot = jax.lax.rem(s, _NBUF)
        in_copy(s, slot).wait()

        @pl.when(s >= _NBUF)
        def _():
            out_copy(s - _NBUF, slot).wait()

        pe3 = pe_ref[pl.ds(0, b * t), :].reshape(b, t, d)
        obuf[slot] = ibuf[slot] + pe3[:, None, :, :]
        out_copy(s, slot).start()

        @pl.when(s + _NBUF < nsteps)
        def _():
            in_copy(s + _NBUF, slot).start()

        @pl.when(s == nsteps - 1)
        def _epilogue():
            for k in range(_NBUF):
                step = nsteps - _NBUF + k
                out_copy(step, step % _NBUF).wait()

    return add_body, nsteps


def kernel(x, frame_indices, frame_embed):
    b, t, p, d = x.shape
    bt = b * t

    align = 8 * _NUM_WORKERS
    bt_pad = ((bt + align - 1) // align) * align
    idx = frame_indices.reshape(bt).astype(jnp.int32)
    idx_pad = jnp.pad(idx, (0, bt_pad - bt))

    pe = _sc_gather(frame_embed, idx_pad, bt_pad // _NUM_WORKERS)

    xt = jnp.swapaxes(x, 1, 2)
    add_body, nsteps = _make_add_body(b, t, p, d)
    out_t = pl.pallas_call(
        add_body,
        grid=(nsteps,),
        in_specs=[
            pl.BlockSpec((bt_pad, d), lambda s: (0, 0)),
            pl.BlockSpec(memory_space=pl.ANY),
        ],
        out_specs=pl.BlockSpec(memory_space=pl.ANY),
        out_shape=jax.ShapeDtypeStruct((b, p, t, d), jnp.float32),
        scratch_shapes=[
            pltpu.VMEM((_NBUF, b, _PCH, t, d), jnp.float32),
            pltpu.VMEM((_NBUF, b, _PCH, t, d), jnp.float32),
            pltpu.SemaphoreType.DMA((_NBUF,)),
            pltpu.SemaphoreType.DMA((_NBUF,)),
        ],
        compiler_params=pltpu.CompilerParams(
            dimension_semantics=("arbitrary",),
            vmem_limit_bytes=63 * 1024 * 1024,
        ),
    )(pe, xt)
    return jnp.swapaxes(out_t, 1, 2)
